# baseline (speedup 1.0000x reference)
import functools

import jax
import jax.numpy as jnp
from jax import lax
from jax.experimental import pallas as pl
from jax.experimental.pallas import tpu as pltpu
from jax.experimental.pallas import tpu_sc as plsc

N_CLASSES = 1_000_000
EMBED = 16
BATCH = 16384

_NC = 2
_NS = 16
_NW = _NC * _NS
_L = 16

_TC_TOTAL = (N_CLASSES + 127) // 128
_SPW = 245
_CW = 20
_NCH = 13
_CELEM = _CW * 128
_A0MAX = _TC_TOTAL - _CW

_LLOC = 1056
_LBK = 128
_GMAX = _LBK // _L
_OUT_PAD = BATCH * EMBED + _NW * EMBED

_mesh = plsc.VectorSubcoreMesh(core_axis_name="c", subcore_axis_name="s")


@functools.partial(
    pl.kernel,
    mesh=_mesh,
    out_type=jax.ShapeDtypeStruct((_OUT_PAD,), jnp.float32),
    scratch_types=dict(
        lab_v=pltpu.VMEM((BATCH,), jnp.int32),
        buf_a=pltpu.VMEM((EMBED, _CELEM), jnp.float32),
        buf_b=pltpu.VMEM((EMBED, _CELEM), jnp.float32),
        xloc=pltpu.VMEM((_LLOC,), jnp.int32),
        jloc=pltpu.VMEM((_LLOC,), jnp.int32),
        xbk=pltpu.VMEM((_NCH, _LBK), jnp.int32),
        jbk=pltpu.VMEM((_NCH, _LBK), jnp.int32),
        slots=pltpu.VMEM((_GMAX * _L * EMBED,), jnp.float32),
        sem=pltpu.SemaphoreType.DMA,
        sem_sc=pltpu.SemaphoreType.DMA,
    ),
    compiler_params=pltpu.CompilerParams(needs_layout_passes=False),
)
def _gather_kernel(tab_hbm, labels_hbm, out_hbm, lab_v, buf_a, buf_b,
                   xloc, jloc, xbk, jbk, slots, sem, sem_sc):
    sc = lax.axis_index("c")
    sid = lax.axis_index("s")
    wid = sc * _NS + sid
    c_lo = wid * _SPW
    lane = lax.iota(jnp.int32, _L)
    bufs = (buf_a, buf_b)

    def chunk_a0(k):
        return jnp.minimum(c_lo + k * _CW, _A0MAX)

    def fire_chunk(k):
        buf = bufs[k % 2]
        o0 = pl.multiple_of(chunk_a0(k) * 128, 128)
        h0 = pltpu.async_copy(tab_hbm.at[pl.ds(0, 8), pl.ds(o0, _CELEM)],
                              buf.at[pl.ds(0, 8), :], sem)
        h1 = pltpu.async_copy(tab_hbm.at[pl.ds(8, 8), pl.ds(o0, _CELEM)],
                              buf.at[pl.ds(8, 8), :], sem)
        return (h0, h1)

    pltpu.sync_copy(labels_hbm, lab_v)
    streams = {0: fire_chunk(0), 1: fire_chunk(1)}

    def scan_body(g4, ptr):
        for u in range(4):
            g = g4 * 4 + u
            lv = lab_v[pl.ds(g * _L, _L)]
            rel = (lv >> 7) - c_lo
            m = jnp.logical_and(rel >= 0, rel < _SPW)
            plsc.store_compressed(xloc.at[pl.ds(ptr, _L)], lv, mask=m)
            plsc.store_compressed(jloc.at[pl.ds(ptr, _L)], g * _L + lane,
                                  mask=m)
            ptr = jnp.minimum(ptr + jnp.sum(m.astype(jnp.int32)),
                              _LLOC - 2 * _L)
        return ptr

    nloc = lax.fori_loop(0, BATCH // _L // 4, scan_body, 0)
    x_sent = jnp.full((_L,), c_lo * 128, jnp.int32)
    j_sent = jnp.full((_L,), BATCH, jnp.int32) + wid
    xloc[pl.ds(nloc, _L)] = x_sent
    jloc[pl.ds(nloc, _L)] = j_sent
    xloc[pl.ds(nloc + _L, _L)] = x_sent
    jloc[pl.ds(nloc + _L, _L)] = j_sent

    def bin_body(g, ptrs):
        xv = xloc[pl.ds(g * _L, _L)]
        jv = jloc[pl.ds(g * _L, _L)]
        kb = ((xv >> 7) - c_lo) // _CW
        new_ptrs = []
        for c in range(_NCH):
            m = jnp.logical_and(kb == c, jv < BATCH)
            plsc.store_compressed(xbk.at[c, pl.ds(ptrs[c], _L)], xv, mask=m)
            plsc.store_compressed(jbk.at[c, pl.ds(ptrs[c], _L)], jv, mask=m)
            new_ptrs.append(jnp.minimum(
                ptrs[c] + jnp.sum(m.astype(jnp.int32)), _LBK - 2 * _L))
        return tuple(new_ptrs)

    nbk = lax.fori_loop(0, (nloc + _L - 1) // _L + 1, bin_body, (0,) * _NCH)
    for c in range(_NCH):
        xb_sent = jnp.full((_L,), chunk_a0(c) * 128, jnp.int32)
        xbk[c, pl.ds(nbk[c], _L)] = xb_sent
        jbk[c, pl.ds(nbk[c], _L)] = j_sent
        xbk[c, pl.ds(nbk[c] + _L, _L)] = xb_sent
        jbk[c, pl.ds(nbk[c] + _L, _L)] = j_sent

    ngroups = [jnp.minimum((nbk[k] + _L - 1) // _L + 1, _GMAX)
               for k in range(_NCH)]
    def drain_body(i, carry):
        pltpu.make_async_copy(out_hbm.at[pl.ds(0, _L * EMBED)],
                              slots.at[pl.ds(0, _L * EMBED)], sem_sc).wait()
        return carry

    for k in range(_NCH):
        for h in streams.pop(k):
            h.wait()
        if k > 0:
            lax.fori_loop(0, ngroups[k - 1], drain_body, 0)

        def ext_body(g, carry, k=k):
            xv = xbk[k, pl.ds(g * _L, _L)]
            jv = jbk[k, pl.ds(g * _L, _L)]
            jv = jnp.clip(jv, 0, BATCH + _NW - 1)
            colv = jnp.clip(xv - chunk_a0(k) * 128, 0, _CELEM - 1)
            sbase = g * _L * EMBED
            for l in range(_L):
                c_l = jnp.broadcast_to(colv[l], (_L,))
                vals = plsc.load_gather(bufs[k % 2], [lane, c_l])
                slots[pl.ds(sbase + l * EMBED, EMBED)] = vals
                pltpu.async_copy(
                    slots.at[pl.ds(sbase + l * EMBED, EMBED)],
                    out_hbm.at[pl.ds(jv[l] * EMBED, EMBED)], sem_sc)
            return carry

        lax.fori_loop(0, ngroups[k], ext_body, 0)
        if k + 2 < _NCH:
            streams[k + 2] = fire_chunk(k + 2)

    lax.fori_loop(0, ngroups[_NCH - 1], drain_body, 0)


def kernel(labels, embed_table):
    flat = _gather_kernel(embed_table.T, labels.astype(jnp.int32))
    return flat.reshape(BATCH + _NW, EMBED)[:BATCH]
